# baseline (speedup 1.0000x reference)
import functools

import jax
import jax.numpy as jnp
from jax import lax
from jax.experimental import pallas as pl
from jax.experimental.pallas import tpu as pltpu
from jax.experimental.pallas import tpu_sc as plsc

_EMBED = 32
_BATCH = 16384
_R_LO, _R_HI = 0.5, 5.0

_info = plsc.get_sparse_core_info()
_NC = _info.num_cores
_NS = _info.num_subcores
_L = _info.num_lanes
_NW = _NC * _NS
_BPW = _BATCH // _NW
_CHUNK = 128
_NCHUNK = _BPW // _CHUNK
_NBLK = _BPW // _L


def _sc_body(uid_hbm, mid_hbm, eu_hbm, bu_hbm, em_hbm, bm_hbm, out_hbm,
             uidx, midx, urows, mrows, ubv, mbv, outv, sem):
    wid = lax.axis_index("s") * _NC + lax.axis_index("c")
    base = wid * _BPW

    pltpu.sync_copy(uid_hbm.at[pl.ds(wid * _NCHUNK, _NCHUNK)], uidx)
    pltpu.sync_copy(mid_hbm.at[pl.ds(wid * _NCHUNK, _NCHUNK)], midx)

    handles = []
    for j in range(_NCHUNK):
        sl = pl.ds(j * _CHUNK, _CHUNK)
        handles.append(pltpu.async_copy(eu_hbm.at[uidx.at[j]], urows.at[sl], sem))
        handles.append(pltpu.async_copy(em_hbm.at[midx.at[j]], mrows.at[sl], sem))
        handles.append(pltpu.async_copy(bu_hbm.at[uidx.at[j]], ubv.at[sl], sem))
        handles.append(pltpu.async_copy(bm_hbm.at[midx.at[j]], mbv.at[sl], sem))
    for h in handles:
        h.wait()

    iota = lax.iota(jnp.int32, _L)

    def blk(b, carry):
        rows = b * _L + iota
        acc = jnp.zeros((_L,), jnp.float32)
        for e in range(_EMBED):
            col = jnp.full((_L,), e, jnp.int32)
            uv = plsc.load_gather(urows, [rows, col])
            mv = plsc.load_gather(mrows, [rows, col])
            acc = acc + uv * mv
        x = acc + ubv[pl.ds(b * _L, _L)] + mbv[pl.ds(b * _L, _L)]
        y = (_R_HI - _R_LO) / (1.0 + jnp.exp(-x)) + _R_LO
        outv[pl.ds(b * _L, _L)] = y
        return carry

    lax.fori_loop(0, _NBLK, blk, 0)

    pltpu.sync_copy(outv, out_hbm.at[pl.ds(base, _BPW)])


@functools.partial(jax.jit, donate_argnums=())
def _run(uid, mid, embed_user, bias_user, embed_movie, bias_movie):
    mesh = plsc.VectorSubcoreMesh(core_axis_name="c", subcore_axis_name="s")
    k = pl.kernel(
        _sc_body,
        out_type=jax.ShapeDtypeStruct((_BATCH,), jnp.float32),
        mesh=mesh,
        compiler_params=pltpu.CompilerParams(
            use_tc_tiling_on_sc=False, needs_layout_passes=False),
        scratch_types=[
            pltpu.VMEM((_NCHUNK, _CHUNK), jnp.int32),
            pltpu.VMEM((_NCHUNK, _CHUNK), jnp.int32),
            pltpu.VMEM((_BPW, _EMBED), jnp.float32),
            pltpu.VMEM((_BPW, _EMBED), jnp.float32),
            pltpu.VMEM((_BPW,), jnp.float32),
            pltpu.VMEM((_BPW,), jnp.float32),
            pltpu.VMEM((_BPW,), jnp.float32),
            pltpu.SemaphoreType.DMA,
        ],
    )
    return k(uid, mid, embed_user, bias_user, embed_movie, bias_movie)


def kernel(inp, embed_user, bias_user, embed_movie, bias_movie):
    uid = inp[:, 0].reshape(_BATCH // _CHUNK, _CHUNK)
    mid = inp[:, 1].reshape(_BATCH // _CHUNK, _CHUNK)
    return _run(uid, mid, embed_user, bias_user.reshape(-1),
                embed_movie, bias_movie.reshape(-1))


# speedup vs baseline: 1.0026x; 1.0026x over previous
import functools

import jax
import jax.numpy as jnp
from jax import lax
from jax.experimental import pallas as pl
from jax.experimental.pallas import tpu as pltpu
from jax.experimental.pallas import tpu_sc as plsc

_EMBED = 32
_BATCH = 16384
_R_LO, _R_HI = 0.5, 5.0

_info = plsc.get_sparse_core_info()
_NC = _info.num_cores
_NS = _info.num_subcores
_L = _info.num_lanes
_NW = _NC * _NS
_BPW = _BATCH // _NW
_CHUNK = 128
_NCHUNK = _BPW // _CHUNK
_NBLK = _BPW // _L


def _sc_body(inp_t, eu_hbm, bu_t, em_hbm, bm_t, out_hbm,
             uidv, midv, uidx, midx, urows, mrows, ubv, mbv, outv, sem):
    wid = lax.axis_index("s") * _NC + lax.axis_index("c")
    base = wid * _BPW

    pltpu.sync_copy(inp_t.at[0, pl.ds(base, _BPW)], uidv)
    pltpu.sync_copy(inp_t.at[1, pl.ds(base, _BPW)], midv)

    for i in range(_BPW // _L):
        uidx[i // 8, pl.ds((i % 8) * _L, _L)] = uidv[pl.ds(i * _L, _L)]
        midx[i // 8, pl.ds((i % 8) * _L, _L)] = midv[pl.ds(i * _L, _L)]

    bu_flat = bu_t.at[0]
    bm_flat = bm_t.at[0]
    handles = []
    for j in range(_NCHUNK):
        sl = pl.ds(j * _CHUNK, _CHUNK)
        handles.append(pltpu.async_copy(eu_hbm.at[uidx.at[j]], urows.at[sl], sem))
        handles.append(pltpu.async_copy(em_hbm.at[midx.at[j]], mrows.at[sl], sem))
        handles.append(pltpu.async_copy(bu_flat.at[uidx.at[j]], ubv.at[sl], sem))
        handles.append(pltpu.async_copy(bm_flat.at[midx.at[j]], mbv.at[sl], sem))
    for h in handles:
        h.wait()

    iota = lax.iota(jnp.int32, _L)

    def blk(b, carry):
        rows = b * _L + iota
        sl = pl.ds(b * _L, _L)
        acc = ubv[sl] + mbv[sl]
        for e in range(_EMBED):
            col = jnp.full((_L,), e, jnp.int32)
            uv = plsc.load_gather(urows, [rows, col])
            mv = plsc.load_gather(mrows, [rows, col])
            acc = acc + uv * mv
        y = (_R_HI - _R_LO) / (1.0 + jnp.exp(-acc)) + _R_LO
        outv[sl] = y
        return carry

    lax.fori_loop(0, _NBLK, blk, 0)

    pltpu.sync_copy(outv, out_hbm.at[pl.ds(base, _BPW)])


@jax.jit
def _run(inp_t, eu, bu_t, em, bm_t):
    mesh = plsc.VectorSubcoreMesh(core_axis_name="c", subcore_axis_name="s")
    k = pl.kernel(
        _sc_body,
        out_type=jax.ShapeDtypeStruct((_BATCH,), jnp.float32),
        mesh=mesh,
        compiler_params=pltpu.CompilerParams(
            use_tc_tiling_on_sc=False, needs_layout_passes=False),
        scratch_types=[
            pltpu.VMEM((_BPW,), jnp.int32),
            pltpu.VMEM((_BPW,), jnp.int32),
            pltpu.VMEM((_NCHUNK, _CHUNK), jnp.int32),
            pltpu.VMEM((_NCHUNK, _CHUNK), jnp.int32),
            pltpu.VMEM((_BPW, _EMBED), jnp.float32),
            pltpu.VMEM((_BPW, _EMBED), jnp.float32),
            pltpu.VMEM((_BPW,), jnp.float32),
            pltpu.VMEM((_BPW,), jnp.float32),
            pltpu.VMEM((_BPW,), jnp.float32),
            pltpu.SemaphoreType.DMA,
        ],
    )
    return k(inp_t, eu, bu_t, em, bm_t)


def kernel(inp, embed_user, bias_user, embed_movie, bias_movie):
    return _run(inp.T, embed_user, bias_user.T, embed_movie, bias_movie.T)
